# baseline (speedup 1.0000x reference)
import functools

import jax
import jax.numpy as jnp
from jax import lax
from jax.experimental import pallas as pl
from jax.experimental.pallas import tpu as pltpu
from jax.experimental.pallas import tpu_sc as plsc

_D = 64
_BW = 128
_BPW = 4
_CG = 8


def _gather_kernel(NB, H):
    info = plsc.get_sparse_core_info()
    NW = info.num_cores * info.num_subcores
    n_blocks = NB // _BW
    assert n_blocks == NW * _BPW and H % 2 == 0
    X = H * 8 * n_blocks

    mesh = plsc.VectorSubcoreMesh(core_axis_name="c", subcore_axis_name="s")

    scratch = (
        [pltpu.VMEM((_BW, H), jnp.int32)]
        + [pltpu.VMEM((_BW,), jnp.int32) for _ in range(2)]
        + [pltpu.VMEM((_BW, _D), jnp.float32) for _ in range(2)]
        + [pltpu.VMEM((_D, _BW), jnp.float32) for _ in range(2)]
        + [pltpu.SemaphoreType.DMA for _ in range(4)]
    )

    @functools.partial(
        pl.kernel,
        mesh=mesh,
        out_type=jax.ShapeDtypeStruct((X, 8, _BW), jnp.float32),
        scratch_types=scratch,
        compiler_params=pltpu.CompilerParams(
            use_tc_tiling_on_sc=False, needs_layout_passes=False),
    )
    def k(table_hbm, idx_hbm, out_hbm, idx_blk, *bufs):
        idx_col = bufs[0:2]
        rows_v = bufs[2:4]
        slab_v = bufs[4:6]
        sem_g = bufs[6:8]
        sem_w = bufs[8:10]

        wid = lax.axis_index("s") * info.num_cores + lax.axis_index("c")
        iota16 = lax.iota(jnp.int32, 16)

        def compact_idx(p, j):
            col = lax.full((16,), 0, jnp.int32) + j
            for g in range(_BW // 16):
                v = plsc.load_gather(idx_blk, [iota16 + (16 * g), col])
                idx_col[p][pl.ds(16 * g, 16)] = v

        def gather_copy(p):
            return pltpu.make_async_copy(
                table_hbm.at[idx_col[p]], rows_v[p], sem_g[p])

        def transpose(p):
            def tbody(cg, carry):
                for ci in range(_CG):
                    c = cg * _CG + ci
                    cc = lax.full((16,), 0, jnp.int32) + c
                    for d0 in range(0, _D, 16):
                        x = rows_v[p][c, pl.ds(d0, 16)]
                        plsc.store_scatter(slab_v[p], [iota16 + d0, cc], x)
                return carry

            lax.fori_loop(0, _BW // _CG, tbody, 0)

        def write_copies(p, j, blk):
            for a in range(8):
                yield pltpu.make_async_copy(
                    slab_v[p].at[pl.ds(8 * a, 8)],
                    out_hbm.at[j * (8 * n_blocks) + a * n_blocks + blk],
                    sem_w[p],
                )

        def fire_writes(p, j, blk):
            for cp in write_copies(p, j, blk):
                cp.start()

        def wait_writes(p, j, blk):
            for cp in write_copies(p, j, blk):
                cp.wait()

        def bbody(bb, carry):
            blk = wid * _BPW + bb
            pltpu.sync_copy(idx_hbm.at[pl.ds(blk * _BW, _BW)], idx_blk)
            for p in range(2):
                compact_idx(p, p)
                gather_copy(p).start()

            def body(j2, carry2):
                for p in range(2):
                    j = j2 * 2 + p
                    gather_copy(p).wait()
                    compact_idx(p, j + 2)
                    @pl.when(j2 > 0)
                    def _():
                        wait_writes(p, j - 2, blk)
                    transpose(p)
                    gather_copy(p).start()
                    fire_writes(p, j, blk)
                return carry2

            lax.fori_loop(0, H // 2 - 1, body, 0)

            for p in range(2):
                j = H - 2 + p
                gather_copy(p).wait()
                wait_writes(p, j - 2, blk)
                transpose(p)
                fire_writes(p, j, blk)
            for p in range(2):
                wait_writes(p, H - 2 + p, blk)
            return carry

        lax.fori_loop(0, _BPW, bbody, 0)

    return k


def kernel(embs, step):
    nb, h = step.shape
    out3 = _gather_kernel(nb, h)(embs, step)
    out5 = out3.reshape(h, 8, nb // _BW, 8, _BW)
    out = out5.transpose(2, 4, 0, 1, 3)
    return out.reshape(nb, h, _D)


# speedup vs baseline: 1.0006x; 1.0006x over previous
import functools

import jax
import jax.numpy as jnp
from jax import lax
from jax.experimental import pallas as pl
from jax.experimental.pallas import tpu as pltpu
from jax.experimental.pallas import tpu_sc as plsc

_D = 64
_BW = 128
_BPW = 4
_CG = 8


def _gather_kernel(NB, H):
    info = plsc.get_sparse_core_info()
    NW = info.num_cores * info.num_subcores
    n_blocks = NB // _BW
    assert n_blocks == NW * _BPW and H % 2 == 0
    X = H * 8 * n_blocks

    mesh = plsc.VectorSubcoreMesh(core_axis_name="c", subcore_axis_name="s")

    scratch = (
        [pltpu.VMEM((_BW, H), jnp.int32)]
        + [pltpu.VMEM((_BW,), jnp.int32) for _ in range(2)]
        + [pltpu.VMEM((_BW, _D), jnp.float32) for _ in range(2)]
        + [pltpu.VMEM((_D * _BW,), jnp.float32) for _ in range(2)]
        + [pltpu.SemaphoreType.DMA for _ in range(4)]
    )

    @functools.partial(
        pl.kernel,
        mesh=mesh,
        out_type=jax.ShapeDtypeStruct((X * 8 * _BW,), jnp.float32),
        scratch_types=scratch,
        compiler_params=pltpu.CompilerParams(
            use_tc_tiling_on_sc=False, needs_layout_passes=False),
    )
    def k(table_hbm, idx_hbm, out_hbm, idx_blk, *bufs):
        idx_col = bufs[0:2]
        rows_v = bufs[2:4]
        slab_v = bufs[4:6]
        sem_g = bufs[6:8]
        sem_w = bufs[8:10]

        wid = lax.axis_index("s") * info.num_cores + lax.axis_index("c")
        iota16 = lax.iota(jnp.int32, 16)

        def compact_idx(p, j):
            col = lax.full((16,), 0, jnp.int32) + j
            for g in range(_BW // 16):
                v = plsc.load_gather(idx_blk, [iota16 + (16 * g), col])
                idx_col[p][pl.ds(16 * g, 16)] = v

        def gather_copy(p):
            return pltpu.make_async_copy(
                table_hbm.at[idx_col[p]], rows_v[p], sem_g[p])

        iotaK = iota16 * _BW

        def transpose(p):
            def tbody(cg, carry):
                for ci in range(_CG):
                    c = cg * _CG + ci
                    base_c = lax.full((16,), 0, jnp.int32) + c
                    for d0 in range(0, _D, 16):
                        x = rows_v[p][c, pl.ds(d0, 16)]
                        plsc.store_scatter(
                            slab_v[p], [iotaK + (d0 * _BW) + base_c], x)
                return carry

            lax.fori_loop(0, _BW // _CG, tbody, 0)

        def write_copies(p, j, blk):
            for a in range(8):
                s = j * (8 * n_blocks) + a * n_blocks + blk
                yield pltpu.make_async_copy(
                    slab_v[p].at[pl.ds(1024 * a, 1024)],
                    out_hbm.at[pl.ds(s * 1024, 1024)],
                    sem_w[p],
                )

        def fire_writes(p, j, blk):
            for cp in write_copies(p, j, blk):
                cp.start()

        def wait_writes(p, j, blk):
            for cp in write_copies(p, j, blk):
                cp.wait()

        def bbody(bb, carry):
            blk = wid * _BPW + bb
            pltpu.sync_copy(idx_hbm.at[pl.ds(blk * _BW, _BW)], idx_blk)
            for p in range(2):
                compact_idx(p, p)
                gather_copy(p).start()

            def body(j2, carry2):
                for p in range(2):
                    j = j2 * 2 + p
                    gather_copy(p).wait()
                    compact_idx(p, j + 2)
                    @pl.when(j2 > 0)
                    def _():
                        wait_writes(p, j - 2, blk)
                    transpose(p)
                    gather_copy(p).start()
                    fire_writes(p, j, blk)
                return carry2

            lax.fori_loop(0, H // 2 - 1, body, 0)

            for p in range(2):
                j = H - 2 + p
                gather_copy(p).wait()
                wait_writes(p, j - 2, blk)
                transpose(p)
                fire_writes(p, j, blk)
            for p in range(2):
                wait_writes(p, H - 2 + p, blk)
            return carry

        lax.fori_loop(0, _BPW, bbody, 0)

    return k


def kernel(embs, step):
    nb, h = step.shape
    out3 = _gather_kernel(nb, h)(embs, step)
    out5 = out3.reshape(h, 8, nb // _BW, 8, _BW)
    out = out5.transpose(2, 4, 0, 1, 3)
    return out.reshape(nb, h, _D)


# speedup vs baseline: 2.5208x; 2.5193x over previous
import functools

import jax
import jax.numpy as jnp
from jax import lax
from jax.experimental import pallas as pl
from jax.experimental.pallas import tpu as pltpu
from jax.experimental.pallas import tpu_sc as plsc

_D = 64
_BW = 128
_BPW = 4
_CG = 8


def _gather_kernel(NB, H):
    info = plsc.get_sparse_core_info()
    NW = info.num_cores * info.num_subcores
    n_blocks = NB // _BW
    assert n_blocks == NW * _BPW and H % 2 == 0
    X = H * 8 * n_blocks

    mesh = plsc.VectorSubcoreMesh(core_axis_name="c", subcore_axis_name="s")

    scratch = (
        [pltpu.VMEM((_BW, H), jnp.int32)]
        + [pltpu.VMEM((_BW,), jnp.int32) for _ in range(2)]
        + [pltpu.VMEM((_BW, _D), jnp.float32) for _ in range(2)]
        + [pltpu.VMEM((_D, _BW + 1), jnp.float32) for _ in range(2)]
        + [pltpu.SemaphoreType.DMA for _ in range(4)]
    )

    @functools.partial(
        pl.kernel,
        mesh=mesh,
        out_type=jax.ShapeDtypeStruct((X, 8, _BW), jnp.float32),
        scratch_types=scratch,
        compiler_params=pltpu.CompilerParams(
            use_tc_tiling_on_sc=False, needs_layout_passes=False),
    )
    def k(table_hbm, idx_hbm, out_hbm, idx_blk, *bufs):
        idx_col = bufs[0:2]
        rows_v = bufs[2:4]
        slab_v = bufs[4:6]
        sem_g = bufs[6:8]
        sem_w = bufs[8:10]

        wid = lax.axis_index("s") * info.num_cores + lax.axis_index("c")
        iota16 = lax.iota(jnp.int32, 16)

        def compact_idx(p, j):
            col = lax.full((16,), 0, jnp.int32) + j
            for g in range(_BW // 16):
                v = plsc.load_gather(idx_blk, [iota16 + (16 * g), col])
                idx_col[p][pl.ds(16 * g, 16)] = v

        def gather_copy(p):
            return pltpu.make_async_copy(
                table_hbm.at[idx_col[p]], rows_v[p], sem_g[p])

        def transpose(p):
            def tbody(cg, carry):
                for ci in range(_CG):
                    c = cg * _CG + ci
                    base_c = lax.full((16,), 0, jnp.int32) + c
                    for d0 in range(0, _D, 16):
                        x = rows_v[p][c, pl.ds(d0, 16)]
                        plsc.store_scatter(
                            slab_v[p], [iota16 + d0, base_c], x)
                return carry

            lax.fori_loop(0, _BW // _CG, tbody, 0)

        def write_copies(p, j, blk):
            for a in range(8):
                s = j * (8 * n_blocks) + a * n_blocks + blk
                yield pltpu.make_async_copy(
                    slab_v[p].at[pl.ds(8 * a, 8), pl.ds(0, _BW)],
                    out_hbm.at[s],
                    sem_w[p],
                )

        def fire_writes(p, j, blk):
            for cp in write_copies(p, j, blk):
                cp.start()

        def wait_writes(p, j, blk):
            for cp in write_copies(p, j, blk):
                cp.wait()

        def bbody(bb, carry):
            blk = wid * _BPW + bb
            pltpu.sync_copy(idx_hbm.at[pl.ds(blk * _BW, _BW)], idx_blk)
            for p in range(2):
                compact_idx(p, p)
                gather_copy(p).start()

            def body(j2, carry2):
                for p in range(2):
                    j = j2 * 2 + p
                    gather_copy(p).wait()
                    compact_idx(p, j + 2)
                    @pl.when(j2 > 0)
                    def _():
                        wait_writes(p, j - 2, blk)
                    transpose(p)
                    gather_copy(p).start()
                    fire_writes(p, j, blk)
                return carry2

            lax.fori_loop(0, H // 2 - 1, body, 0)

            for p in range(2):
                j = H - 2 + p
                gather_copy(p).wait()
                wait_writes(p, j - 2, blk)
                transpose(p)
                fire_writes(p, j, blk)
            for p in range(2):
                wait_writes(p, H - 2 + p, blk)
            return carry

        lax.fori_loop(0, _BPW, bbody, 0)

    return k


def kernel(embs, step):
    nb, h = step.shape
    out3 = _gather_kernel(nb, h)(embs, step)
    out5 = out3.reshape(h, 8, nb // _BW, 8, _BW)
    out = out5.transpose(2, 4, 0, 1, 3)
    return out.reshape(nb, h, _D)
